# baseline (speedup 1.0000x reference)
import functools

import jax
import jax.numpy as jnp
from jax import lax
from jax.experimental import pallas as pl
from jax.experimental.pallas import tpu as pltpu
from jax.experimental.pallas import tpu_sc as plsc

NC = 2
NS = 16
NW = NC * NS

B, T, F = 16, 1024, 768
ROWS = B * T
RPW = ROWS // NW
CHUNK = 16
NCH = RPW // CHUNK
NB = 4
CHW = CHUNK * F
VL = 16

_K2 = -2.0 * 0.7978845608028654
_A = 0.044715


def _gelu_vec(x):
    inner = x + _A * x * x * x
    return x / (1.0 + jnp.exp(_K2 * inner))


def _sc_body(hid, idx, out, bufs, mask_v, idx_v, gsems, ssems):
    wid = lax.axis_index("s") * NC + lax.axis_index("c")
    batch = wid // NC
    half = wid % NC
    wbase = wid * RPW
    t0 = half * (T // NC)

    def g_copy0(c, q):
        fstart = (wbase + c * CHUNK) * F
        return pltpu.make_async_copy(hid.at[pl.ds(fstart, CHW)], bufs[q],
                                     gsems[q])

    g_copy0(0, 0).start()

    pltpu.sync_copy(idx, idx_v)
    zeros = jnp.zeros((VL,), jnp.float32)
    ones = jnp.ones((VL,), jnp.float32)
    for k in range(RPW // VL):
        mask_v[pl.ds(k * VL, VL)] = zeros
    nidx = idx.shape[0]
    for k in range(nidx // VL):
        iv = idx_v[pl.ds(k * VL, VL)] - t0
        inb = (iv >= 0) & (iv < RPW)
        ivc = jnp.clip(iv, 0, RPW - 1)
        plsc.store_scatter(mask_v, [ivc], ones, mask=inb)

    def g_copy(c, q):
        fstart = (wbase + c * CHUNK) * F
        return pltpu.make_async_copy(hid.at[pl.ds(fstart, CHW)], bufs[q],
                                     gsems[q])

    def s_copy(c, q):
        fstart = (wbase + c * CHUNK) * F
        return pltpu.make_async_copy(bufs[q], out.at[pl.ds(fstart, CHW)],
                                     ssems[q])

    def chunk_step(c, q):
        qn = [(q + 1) % NB]

        @pl.when(c >= NB - 1)
        def _():
            s_copy(c - (NB - 1), qn[0]).wait()

        @pl.when(c + 1 < NCH)
        def _():
            g_copy(c + 1, qn[0]).start()

        g_copy(c, q).wait()

        buf = bufs[q]

        def row_step(j, _):
            tloc = c * CHUNK + j
            m = plsc.load_gather(mask_v, [jnp.full((VL,), tloc, jnp.int32)])
            sel = jnp.max(m, axis=0) > 0.5

            @pl.when(sel)
            def _():
                base = j * F
                for v in range(F // VL):
                    sl = pl.ds(base + v * VL, VL)
                    buf[sl] = _gelu_vec(buf[sl])

            return 0

        lax.fori_loop(0, CHUNK, row_step, 0)
        s_copy(c, q).start()

    def outer(g, _):
        for k in range(NB):
            chunk_step(g * NB + k, k)
        return 0

    lax.fori_loop(0, NCH // NB, outer, 0)

    for k in range(1, NB):
        s_copy(NCH - NB + k, k).wait()


@functools.partial(
    pl.kernel,
    out_type=jax.ShapeDtypeStruct((ROWS * F,), jnp.float32),
    mesh=plsc.VectorSubcoreMesh(core_axis_name="c", subcore_axis_name="s"),
    compiler_params=pltpu.CompilerParams(needs_layout_passes=False),
    scratch_types=[
        [pltpu.VMEM((CHW,), jnp.float32) for _ in range(NB)],
        pltpu.VMEM((RPW,), jnp.float32),
        pltpu.VMEM((512,), jnp.int32),
        [pltpu.SemaphoreType.DMA for _ in range(NB)],
        [pltpu.SemaphoreType.DMA for _ in range(NB)],
    ],
)
def _sc_kernel(hid, idx, out, bufs, mask_v, idx_v, gsems, ssems):
    _sc_body(hid, idx, out, bufs, mask_v, idx_v, gsems, ssems)


def kernel(hidden_states, true_indices):
    flat = jnp.reshape(hidden_states, (-1,))
    out = _sc_kernel(flat, true_indices)
    return jnp.reshape(out, hidden_states.shape)


# speedup vs baseline: 2.0681x; 2.0681x over previous
import functools

import jax
import jax.numpy as jnp
from jax import lax
from jax.experimental import pallas as pl
from jax.experimental.pallas import tpu as pltpu
from jax.experimental.pallas import tpu_sc as plsc

NC = 2
NS = 16
NW = NC * NS

B, T, F = 16, 1024, 768
TPW = T // NC
CHUNK = 16
NCH = TPW // CHUNK
NB = 3
VL = 16

_K2 = -2.0 * 0.7978845608028654
_A = 0.044715


def _gelu_vec(x):
    inner = x + _A * x * x * x
    return x / (1.0 + jnp.exp(_K2 * inner))


def _sc_body(hid, idx, out, ibufs, obufs, mask_v, idx_v, gsems, ssems):
    wid = lax.axis_index("s") * NC + lax.axis_index("c")
    batch = wid // NC
    t0 = (wid % NC) * TPW

    def g_copy(c, q):
        return pltpu.make_async_copy(
            hid.at[batch, pl.ds(t0 + c * CHUNK, CHUNK)], ibufs[q], gsems[q])

    def s_copy(c, q):
        return pltpu.make_async_copy(
            obufs[q], out.at[batch, pl.ds(t0 + c * CHUNK, CHUNK)], ssems[q])

    for k in range(NB):
        g_copy(k, k).start()

    pltpu.sync_copy(idx, idx_v)
    zeros = jnp.zeros((VL,), jnp.float32)
    ones = jnp.ones((VL,), jnp.float32)
    for k in range(TPW // VL):
        mask_v[pl.ds(k * VL, VL)] = zeros
    nidx = idx.shape[0]
    for k in range(nidx // VL):
        iv = idx_v[pl.ds(k * VL, VL)] - t0
        inb = (iv >= 0) & (iv < TPW)
        ivc = jnp.clip(iv, 0, TPW - 1)
        plsc.store_scatter(mask_v, [ivc], ones, mask=inb)

    def chunk_step(c, q):
        g_copy(c, q).wait()

        @pl.when(c >= NB)
        def _():
            s_copy(c - NB, q).wait()

        ibuf, obuf = ibufs[q], obufs[q]

        def row_step(j, _):
            tloc = c * CHUNK + j
            m = plsc.load_gather(mask_v, [jnp.full((VL,), tloc, jnp.int32)])
            sel = jnp.max(m, axis=0) > 0.5

            @pl.when(sel)
            def _():
                for v in range(F // VL):
                    sl = pl.ds(v * VL, VL)
                    obuf[j, sl] = _gelu_vec(ibuf[j, sl])

            @pl.when(jnp.logical_not(sel))
            def _():
                for v in range(F // VL):
                    sl = pl.ds(v * VL, VL)
                    obuf[j, sl] = ibuf[j, sl]

            return 0

        lax.fori_loop(0, CHUNK, row_step, 0)
        s_copy(c, q).start()

        @pl.when(c + NB < NCH)
        def _():
            g_copy(c + NB, q).start()

    def outer(g, _):
        for k in range(NB):
            chunk_step(g * NB + k, k)
        return 0

    lax.fori_loop(0, NCH // NB, outer, 0)
    for c in range(NCH - NCH % NB, NCH):
        chunk_step(c, c % NB)

    for c in range(NCH - NB, NCH):
        s_copy(c, c % NB).wait()


@functools.partial(
    pl.kernel,
    out_type=jax.ShapeDtypeStruct((B, T, F), jnp.float32),
    mesh=plsc.VectorSubcoreMesh(core_axis_name="c", subcore_axis_name="s"),
    compiler_params=pltpu.CompilerParams(needs_layout_passes=False),
    scratch_types=[
        [pltpu.VMEM((CHUNK, F), jnp.float32) for _ in range(NB)],
        [pltpu.VMEM((CHUNK, F), jnp.float32) for _ in range(NB)],
        pltpu.VMEM((TPW,), jnp.float32),
        pltpu.VMEM((512,), jnp.int32),
        [pltpu.SemaphoreType.DMA for _ in range(NB)],
        [pltpu.SemaphoreType.DMA for _ in range(NB)],
    ],
)
def _sc_kernel(hid, idx, out, ibufs, obufs, mask_v, idx_v, gsems, ssems):
    _sc_body(hid, idx, out, ibufs, obufs, mask_v, idx_v, gsems, ssems)


def kernel(hidden_states, true_indices):
    return _sc_kernel(hidden_states, true_indices)


# speedup vs baseline: 4.0071x; 1.9376x over previous
import functools

import jax
import jax.numpy as jnp
from jax import lax
from jax.experimental import pallas as pl
from jax.experimental.pallas import tpu as pltpu
from jax.experimental.pallas import tpu_sc as plsc

NC = 2
NS = 16
NW = NC * NS

B, T, F = 16, 1024, 768
TPW = T // NC
CHUNK = 16
NCH = TPW // CHUNK
NB = 3
VL = 16

_K2 = -2.0 * 0.7978845608028654
_A = 0.044715


def _gelu_vec(x):
    inner = x + _A * x * x * x
    return x / (1.0 + jnp.exp(_K2 * inner))


def _sc_body(hid, idx, out, ibufs, obufs, mask_v, idx_v, gsems, ssems):
    wid = lax.axis_index("s") * NC + lax.axis_index("c")
    batch = wid // NC
    t0 = (wid % NC) * TPW

    def g_copy(c, q):
        return pltpu.make_async_copy(
            hid.at[batch, pl.ds(t0 + c * CHUNK, CHUNK)], ibufs[q], gsems[q])

    def s_copy(c, q):
        return pltpu.make_async_copy(
            obufs[q], out.at[batch, pl.ds(t0 + c * CHUNK, CHUNK)], ssems[q])

    for k in range(NB):
        g_copy(k, k).start()

    pltpu.sync_copy(idx, idx_v)
    zeros = jnp.zeros((VL,), jnp.float32)
    ones = jnp.ones((VL,), jnp.float32)
    for k in range(TPW // VL):
        mask_v[pl.ds(k * VL, VL)] = zeros
    nidx = idx.shape[0]
    for k in range(nidx // VL):
        iv = idx_v[pl.ds(k * VL, VL)] - t0
        inb = (iv >= 0) & (iv < TPW)
        ivc = jnp.clip(iv, 0, TPW - 1)
        plsc.store_scatter(mask_v, [ivc], ones, mask=inb)

    def chunk_step(c, q):
        g_copy(c, q).wait()

        @pl.when(c >= NB)
        def _():
            s_copy(c - NB, q).wait()

        ibuf, obuf = ibufs[q], obufs[q]

        def row_step(j, _):
            tloc = c * CHUNK + j
            m = plsc.load_gather(mask_v, [jnp.full((VL,), tloc, jnp.int32)])
            mb = m > 0.5

            @plsc.parallel_loop(0, F, VL, unroll=8)
            def _(v):
                sl = pl.ds(v, VL)
                x = ibuf[j, sl]
                obuf[j, sl] = jnp.where(mb, _gelu_vec(x), x)

            return 0

        lax.fori_loop(0, CHUNK, row_step, 0)
        s_copy(c, q).start()

        @pl.when(c + NB < NCH)
        def _():
            g_copy(c + NB, q).start()

    def outer(g, _):
        for k in range(NB):
            chunk_step(g * NB + k, k)
        return 0

    lax.fori_loop(0, NCH // NB, outer, 0)
    for c in range(NCH - NCH % NB, NCH):
        chunk_step(c, c % NB)

    for c in range(NCH - NB, NCH):
        s_copy(c, c % NB).wait()


@functools.partial(
    pl.kernel,
    out_type=jax.ShapeDtypeStruct((B, T, F), jnp.float32),
    mesh=plsc.VectorSubcoreMesh(core_axis_name="c", subcore_axis_name="s"),
    compiler_params=pltpu.CompilerParams(needs_layout_passes=False),
    scratch_types=[
        [pltpu.VMEM((CHUNK, F), jnp.float32) for _ in range(NB)],
        [pltpu.VMEM((CHUNK, F), jnp.float32) for _ in range(NB)],
        pltpu.VMEM((TPW,), jnp.float32),
        pltpu.VMEM((512,), jnp.int32),
        [pltpu.SemaphoreType.DMA for _ in range(NB)],
        [pltpu.SemaphoreType.DMA for _ in range(NB)],
    ],
)
def _sc_kernel(hid, idx, out, ibufs, obufs, mask_v, idx_v, gsems, ssems):
    _sc_body(hid, idx, out, ibufs, obufs, mask_v, idx_v, gsems, ssems)


def kernel(hidden_states, true_indices):
    return _sc_kernel(hidden_states, true_indices)


# speedup vs baseline: 4.1219x; 1.0287x over previous
import functools

import jax
import jax.numpy as jnp
from jax import lax
from jax.experimental import pallas as pl
from jax.experimental.pallas import tpu as pltpu
from jax.experimental.pallas import tpu_sc as plsc

NC = 2
NS = 16
NW = NC * NS

B, T, F = 16, 1024, 768
TPW = T // NC
CHUNK = 16
NCH = TPW // CHUNK
NB = 3
VL = 16

_K2 = -2.0 * 0.7978845608028654
_A = 0.044715


def _gelu_vec(x):
    x2 = x * x
    arg = (_K2 * x) * (1.0 + _A * x2)
    return x / (1.0 + jnp.exp(arg))


def _sc_body(hid, idx, out, ibufs, obufs, mask_v, idx_v, gsems, ssems):
    wid = lax.axis_index("s") * NC + lax.axis_index("c")
    batch = wid // NC
    t0 = (wid % NC) * TPW

    def g_copy(c, q):
        return pltpu.make_async_copy(
            hid.at[batch, pl.ds(t0 + c * CHUNK, CHUNK)], ibufs[q], gsems[q])

    def s_copy(c, q):
        return pltpu.make_async_copy(
            obufs[q], out.at[batch, pl.ds(t0 + c * CHUNK, CHUNK)], ssems[q])

    for k in range(NB):
        g_copy(k, k).start()

    pltpu.sync_copy(idx, idx_v)
    zeros = jnp.zeros((VL,), jnp.float32)
    ones = jnp.ones((VL,), jnp.float32)
    for k in range(TPW // VL):
        mask_v[pl.ds(k * VL, VL)] = zeros
    nidx = idx.shape[0]
    for k in range(nidx // VL):
        iv = idx_v[pl.ds(k * VL, VL)] - t0
        inb = (iv >= 0) & (iv < TPW)
        ivc = jnp.clip(iv, 0, TPW - 1)
        plsc.store_scatter(mask_v, [ivc], ones, mask=inb)

    def chunk_step(c, q):
        g_copy(c, q).wait()

        @pl.when(c >= NB)
        def _():
            s_copy(c - NB, q).wait()

        ibuf, obuf = ibufs[q], obufs[q]

        def row_step(j, _):
            tloc = c * CHUNK + j
            m = plsc.load_gather(mask_v, [jnp.full((VL,), tloc, jnp.int32)])
            sel = jnp.max(m, axis=0) > 0.5

            @pl.when(sel)
            def _():
                @plsc.parallel_loop(0, F, VL, unroll=12)
                def _(v):
                    sl = pl.ds(v, VL)
                    obuf[j, sl] = _gelu_vec(ibuf[j, sl])

            @pl.when(jnp.logical_not(sel))
            def _():
                @plsc.parallel_loop(0, F, VL, unroll=12)
                def _(v):
                    sl = pl.ds(v, VL)
                    obuf[j, sl] = ibuf[j, sl]

            return 0

        lax.fori_loop(0, CHUNK, row_step, 0)
        s_copy(c, q).start()

        @pl.when(c + NB < NCH)
        def _():
            g_copy(c + NB, q).start()

    def outer(g, _):
        for k in range(NB):
            chunk_step(g * NB + k, k)
        return 0

    lax.fori_loop(0, NCH // NB, outer, 0)
    for c in range(NCH - NCH % NB, NCH):
        chunk_step(c, c % NB)

    for c in range(NCH - NB, NCH):
        s_copy(c, c % NB).wait()


@functools.partial(
    pl.kernel,
    out_type=jax.ShapeDtypeStruct((B, T, F), jnp.float32),
    mesh=plsc.VectorSubcoreMesh(core_axis_name="c", subcore_axis_name="s"),
    compiler_params=pltpu.CompilerParams(needs_layout_passes=False),
    scratch_types=[
        [pltpu.VMEM((CHUNK, F), jnp.float32) for _ in range(NB)],
        [pltpu.VMEM((CHUNK, F), jnp.float32) for _ in range(NB)],
        pltpu.VMEM((TPW,), jnp.float32),
        pltpu.VMEM((512,), jnp.int32),
        [pltpu.SemaphoreType.DMA for _ in range(NB)],
        [pltpu.SemaphoreType.DMA for _ in range(NB)],
    ],
)
def _sc_kernel(hid, idx, out, ibufs, obufs, mask_v, idx_v, gsems, ssems):
    _sc_body(hid, idx, out, ibufs, obufs, mask_v, idx_v, gsems, ssems)


def kernel(hidden_states, true_indices):
    return _sc_kernel(hidden_states, true_indices)


# speedup vs baseline: 5.5599x; 1.3489x over previous
import functools

import jax
import jax.numpy as jnp
from jax import lax
from jax.experimental import pallas as pl
from jax.experimental.pallas import tpu as pltpu
from jax.experimental.pallas import tpu_sc as plsc

NC = 2
NS = 16
NW = NC * NS

B, T, F = 16, 1024, 768
TPW = T // NC
CHUNK = 32
NCH = TPW // CHUNK
NB = 4
VL = 16

_K2 = -2.0 * 0.7978845608028654
_A = 0.044715


def _gelu_vec(x):
    x2 = x * x
    arg = (_K2 * x) * (1.0 + _A * x2)
    return x / (1.0 + jnp.exp(arg))


def _sc_body(hid, idx, out, bufs, mask_v, idx_v, gsems, ssems):
    wid = lax.axis_index("s") * NC + lax.axis_index("c")
    batch = wid // NC
    t0 = (wid % NC) * TPW

    def g_copy(c, q):
        return pltpu.make_async_copy(
            hid.at[batch, pl.ds(t0 + c * CHUNK, CHUNK)], bufs[q], gsems[q])

    def s_copy(c, q):
        return pltpu.make_async_copy(
            bufs[q], out.at[batch, pl.ds(t0 + c * CHUNK, CHUNK)], ssems[q])

    g_copy(0, 0).start()

    pltpu.sync_copy(idx, idx_v)
    zeros = jnp.zeros((VL,), jnp.float32)
    ones = jnp.ones((VL,), jnp.float32)
    for k in range(TPW // VL):
        mask_v[pl.ds(k * VL, VL)] = zeros
    nidx = idx.shape[0]
    for k in range(nidx // VL):
        iv = idx_v[pl.ds(k * VL, VL)] - t0
        inb = (iv >= 0) & (iv < TPW)
        ivc = jnp.clip(iv, 0, TPW - 1)
        plsc.store_scatter(mask_v, [ivc], ones, mask=inb)

    def chunk_step(c, q):
        qn = (q + 1) % NB

        @pl.when(c >= NB - 1)
        def _():
            s_copy(c - (NB - 1), qn).wait()

        @pl.when(c + 1 < NCH)
        def _():
            g_copy(c + 1, qn).start()

        g_copy(c, q).wait()
        buf = bufs[q]

        def row_step(j, _):
            tloc = c * CHUNK + j
            m = plsc.load_gather(mask_v, [jnp.full((VL,), tloc, jnp.int32)])
            sel = jnp.max(m, axis=0) > 0.5

            @pl.when(sel)
            def _():
                @plsc.parallel_loop(0, F, VL, unroll=12)
                def _(v):
                    sl = pl.ds(v, VL)
                    buf[j, sl] = _gelu_vec(buf[j, sl])

            return 0

        lax.fori_loop(0, CHUNK, row_step, 0)
        s_copy(c, q).start()

    def outer(g, _):
        for k in range(NB):
            chunk_step(g * NB + k, k)
        return 0

    lax.fori_loop(0, NCH // NB, outer, 0)

    for k in range(1, NB):
        s_copy(NCH - NB + k, k).wait()


@functools.partial(
    pl.kernel,
    out_type=jax.ShapeDtypeStruct((B, T, F), jnp.float32),
    mesh=plsc.VectorSubcoreMesh(core_axis_name="c", subcore_axis_name="s"),
    compiler_params=pltpu.CompilerParams(needs_layout_passes=False),
    scratch_types=[
        [pltpu.VMEM((CHUNK, F), jnp.float32) for _ in range(NB)],
        pltpu.VMEM((TPW,), jnp.float32),
        pltpu.VMEM((512,), jnp.int32),
        [pltpu.SemaphoreType.DMA for _ in range(NB)],
        [pltpu.SemaphoreType.DMA for _ in range(NB)],
    ],
)
def _sc_kernel(hid, idx, out, bufs, mask_v, idx_v, gsems, ssems):
    _sc_body(hid, idx, out, bufs, mask_v, idx_v, gsems, ssems)


def kernel(hidden_states, true_indices):
    return _sc_kernel(hidden_states, true_indices)
